# baseline (speedup 1.0000x reference)
import functools

import jax
import jax.numpy as jnp
from jax.experimental import pallas as pl
from jax.experimental.pallas import tpu as pltpu

_MXU_DT = jnp.float8_e4m3fn


def _body(q_ref, k_ref, o_ref, qa_ref, ka_ref, acc_ref, qs_ref,
          *, inv_d, bm, nsteps):
    m = pl.program_id(1)

    @pl.when(m == 0)
    def _prep():
        q = q_ref[...]
        qn2 = jnp.sum(q * q, axis=1, keepdims=True)
        inv = 1.0 / jnp.maximum(jnp.sqrt(qn2), 1e-12)
        qs_ref[...] = qn2 * inv * inv
        lane_q = jax.lax.broadcasted_iota(jnp.int32, q.shape, 1)
        qa_ref[:, : q.shape[1]] = (q * (16.0 * inv)).astype(_MXU_DT)
        qa_ref[:, q.shape[1]:] = jnp.where(lane_q < 4, 64.0, 0.0).astype(_MXU_DT)

        k = k_ref[...]
        k2 = jnp.sum(k * k, axis=1, keepdims=True)
        res = k2 - float(k.shape[1])
        cols = []
        for _ in range(4):
            c = res.astype(_MXU_DT).astype(jnp.float32)
            cols.append(c)
            res = res - c
        lane_k = jax.lax.broadcasted_iota(jnp.int32, k.shape, 1)
        k2_col = jnp.where(
            lane_k == 0, cols[0],
            jnp.where(lane_k == 1, cols[1],
                      jnp.where(lane_k == 2, cols[2],
                                jnp.where(lane_k == 3, cols[3], 0.0))))
        ka_ref[:, : k.shape[1]] = (-8.0 * k).astype(_MXU_DT)
        ka_ref[:, k.shape[1]:] = k2_col.astype(_MXU_DT)

    t = jax.lax.dot_general(
        qa_ref[...], ka_ref[pl.ds(m * bm, bm), :], (((1,), (1,)), ((), ())),
        preferred_element_type=jnp.float32,
    )
    nchunk = bm // 128
    mins = [t[:, c * 128:(c + 1) * 128] for c in range(nchunk)]
    while len(mins) > 1:
        mins = [jnp.minimum(mins[i], mins[i + 1]) for i in range(0, len(mins), 2)]
    tm = mins[0]

    @pl.when(m == 0)
    def _init():
        acc_ref[...] = tm

    @pl.when(m != 0)
    def _acc():
        acc_ref[...] = jnp.minimum(acc_ref[...], tm)

    @pl.when(m == nsteps - 1)
    def _finalize():
        r = jnp.min(acc_ref[...], axis=1, keepdims=True)
        o_ref[...] = (r * (1.0 / 64.0) + qs_ref[...] + (1.0 / inv_d)) * inv_d


def kernel(query, keys, train):
    B, C, H, W = query.shape
    M, D = keys.shape
    N = B * H * W
    qf = jnp.transpose(query, (0, 2, 3, 1)).reshape(N, C)
    NSPLIT = 2
    BN = N // NSPLIT
    BM = 512
    nsteps = M // BM
    out = pl.pallas_call(
        functools.partial(_body, inv_d=1.0 / D, bm=BM, nsteps=nsteps),
        grid=(NSPLIT, nsteps),
        in_specs=[
            pl.BlockSpec((BN, D), lambda n, m: (n, 0)),
            pl.BlockSpec((M, D), lambda n, m: (0, 0)),
        ],
        out_specs=pl.BlockSpec((BN, 1), lambda n, m: (n, 0)),
        out_shape=jax.ShapeDtypeStruct((N, 1), jnp.float32),
        scratch_shapes=[
            pltpu.VMEM((BN, 2 * D), _MXU_DT),
            pltpu.VMEM((M, 2 * D), _MXU_DT),
            pltpu.VMEM((BN, D), jnp.float32),
            pltpu.VMEM((BN, 1), jnp.float32),
        ],
        compiler_params=pltpu.CompilerParams(
            dimension_semantics=("parallel", "arbitrary"),
        ),
    )(qf, keys)
    return out.reshape(B, H, W)


# speedup vs baseline: 1.1272x; 1.1272x over previous
import functools

import jax
import jax.numpy as jnp
from jax.experimental import pallas as pl
from jax.experimental.pallas import tpu as pltpu

_MXU_DT = jnp.float8_e4m3fn


def _body(q_ref, k_ref, o_ref, qa_ref, ka_ref, acc_ref, qs_ref,
          *, inv_d, bm, nsteps):
    m = pl.program_id(1)

    @pl.when(m == 0)
    def _prep():
        q = q_ref[...]
        qn2 = jnp.sum(q * q, axis=1, keepdims=True)
        inv = 1.0 / jnp.maximum(jnp.sqrt(qn2), 1e-12)
        qs_ref[...] = qn2 * inv * inv
        lane_q = jax.lax.broadcasted_iota(jnp.int32, q.shape, 1)
        qa_ref[:, : q.shape[1]] = (q * (16.0 * inv)).astype(_MXU_DT)
        qa_ref[:, q.shape[1]:] = jnp.where(lane_q < 4, 64.0, 0.0).astype(_MXU_DT)

        k = k_ref[...]
        k2 = jnp.sum(k * k, axis=1, keepdims=True)
        res = k2 - float(k.shape[1])
        cols = []
        for _ in range(4):
            c = res.astype(_MXU_DT).astype(jnp.float32)
            cols.append(c)
            res = res - c
        lane_k = jax.lax.broadcasted_iota(jnp.int32, k.shape, 1)
        k2_col = jnp.where(
            lane_k == 0, cols[0],
            jnp.where(lane_k == 1, cols[1],
                      jnp.where(lane_k == 2, cols[2],
                                jnp.where(lane_k == 3, cols[3], 0.0))))
        ka_ref[:, : k.shape[1]] = (-8.0 * k).astype(_MXU_DT)
        ka_ref[:, k.shape[1]:] = k2_col.astype(_MXU_DT)

    t = jax.lax.dot_general(
        qa_ref[...], ka_ref[pl.ds(m * bm, bm), :], (((1,), (1,)), ((), ())),
        preferred_element_type=jnp.float32,
    )
    nchunk = bm // 128
    mins = [t[:, c * 128:(c + 1) * 128] for c in range(nchunk)]
    while len(mins) > 1:
        mins = [jnp.minimum(mins[i], mins[i + 1]) for i in range(0, len(mins), 2)]
    tm = mins[0]

    @pl.when(m == 0)
    def _init():
        acc_ref[...] = tm

    @pl.when(m != 0)
    def _acc():
        acc_ref[...] = jnp.minimum(acc_ref[...], tm)

    @pl.when(m == nsteps - 1)
    def _finalize():
        r = jnp.min(acc_ref[...], axis=1, keepdims=True)
        o_ref[...] = (r * (1.0 / 64.0) + qs_ref[...] + (1.0 / inv_d)) * inv_d


def kernel(query, keys, train):
    B, C, H, W = query.shape
    M, D = keys.shape
    N = B * H * W
    qf = jnp.transpose(query, (0, 2, 3, 1)).reshape(N, C)
    NSPLIT = 1
    BN = N // NSPLIT
    BM = 512
    nsteps = M // BM
    out = pl.pallas_call(
        functools.partial(_body, inv_d=1.0 / D, bm=BM, nsteps=nsteps),
        grid=(NSPLIT, nsteps),
        in_specs=[
            pl.BlockSpec((BN, D), lambda n, m: (n, 0)),
            pl.BlockSpec((M, D), lambda n, m: (0, 0)),
        ],
        out_specs=pl.BlockSpec((BN, 1), lambda n, m: (n, 0)),
        out_shape=jax.ShapeDtypeStruct((N, 1), jnp.float32),
        scratch_shapes=[
            pltpu.VMEM((BN, 2 * D), _MXU_DT),
            pltpu.VMEM((M, 2 * D), _MXU_DT),
            pltpu.VMEM((BN, D), jnp.float32),
            pltpu.VMEM((BN, 1), jnp.float32),
        ],
        compiler_params=pltpu.CompilerParams(
            dimension_semantics=("parallel", "arbitrary"),
        ),
    )(qf, keys)
    return out.reshape(B, H, W)
